# baseline (speedup 1.0000x reference)
import functools

import jax
import jax.numpy as jnp
from jax import lax
from jax.experimental import pallas as pl
from jax.experimental.pallas import tpu as pltpu
from jax.experimental.pallas import tpu_sc as plsc

_NUM_GSPS = 360


def _sc_embedding_gather(table, row_idx):
    (B,) = row_idx.shape
    D = table.shape[1]
    info = plsc.get_sparse_core_info()
    nw = info.num_cores * info.num_subcores
    L = info.num_lanes
    b_per_w = B // nw
    mesh = plsc.VectorSubcoreMesh(core_axis_name="c", subcore_axis_name="s")

    @functools.partial(
        pl.kernel,
        mesh=mesh,
        out_type=jax.ShapeDtypeStruct((B, D), jnp.float32),
        scratch_types=[
            pltpu.VMEM((b_per_w,), jnp.int32),
            pltpu.VMEM((b_per_w, D), jnp.float32),
            pltpu.SemaphoreType.DMA,
        ],
        compiler_params=pltpu.CompilerParams(use_tc_tiling_on_sc=False),
    )
    def gather_kernel(table_hbm, idx_hbm, out_hbm, idx_v, rows_v, sem):
        wid = lax.axis_index("s") * info.num_cores + lax.axis_index("c")
        base = wid * b_per_w
        pltpu.sync_copy(idx_hbm.at[pl.ds(base, b_per_w)], idx_v)
        for i in range(b_per_w // L):
            sl = pl.ds(i * L, L)
            idx_v[sl] = idx_v[sl] + _NUM_GSPS
        pltpu.async_copy(table_hbm.at[idx_v], rows_v, sem).wait()
        pltpu.sync_copy(rows_v, out_hbm.at[pl.ds(base, b_per_w)])

    return gather_kernel(table, row_idx)


def _assembly_body(T, F, D, t0_ref, az_ref, el_ref, tf_ref, tf0_ref, y_ref,
                   x_ref, emb_ref, pv_ref, out_ref, s_ref):
    C = out_ref.shape[1]
    P = out_ref.shape[2]
    tv_lo = 1 + 2 * F
    tv_hi = 3 + 4 * F
    nan = jnp.float32(jnp.nan)

    s_ref[0:1] = jnp.zeros((1, P), jnp.float32)
    s_ref[1:1 + F] = jnp.transpose(y_ref[0], (1, 0))
    s_ref[1 + F:tv_lo] = jnp.transpose(x_ref[0], (1, 0))
    s_ref[tv_lo:tv_lo + F] = jnp.zeros((F, P), jnp.float32)
    s_ref[tv_lo + F:tv_hi - 2] = jnp.concatenate(
        [jnp.full((1, P), tf0_ref[0, 0, j], jnp.float32) for j in range(F)],
        axis=0)
    s_ref[tv_hi - 2:tv_hi] = jnp.zeros((2, P), jnp.float32)
    s_ref[tv_hi:tv_hi + D] = jnp.transpose(emb_ref[0], (1, 0))
    s_ref[C - 1:C] = jnp.zeros((1, P), jnp.float32)

    S = s_ref[...]
    t0v = t0_ref[0, 0]
    for t in range(T):
        out_ref[t] = S
        bad = t > t0v
        tf_rows = jnp.concatenate(
            [jnp.full((1, P), jnp.where(bad, nan, tf_ref[0, t, j]),
                      jnp.float32) for j in range(F)], axis=0)
        out_ref[t, tv_lo:tv_lo + F] = tf_rows
        azel = jnp.concatenate([
            jnp.full((1, P), jnp.where(bad, nan, az_ref[0, 0, t]),
                     jnp.float32),
            jnp.full((1, P), jnp.where(bad, nan, el_ref[0, 0, t]),
                     jnp.float32)], axis=0)
        out_ref[t, tv_hi - 2:tv_hi] = azel
        out_ref[t, C - 1:C] = jnp.where(
            bad, nan, pv_ref[0, t].reshape(1, P))


def kernel(pv, pv_solar_azimuth, pv_solar_elevation, pv_time_utc_fourier,
           pv_time_utc_fourier_t0, pv_y_osgb_fourier, pv_x_osgb_fourier,
           pv_system_row_number, pv_t0_idx, embedding_table):
    E, T, P = pv.shape
    F = pv_time_utc_fourier.shape[-1]
    D = embedding_table.shape[1]
    C = 3 + 4 * F + D + 1

    emb = _sc_embedding_gather(
        embedding_table,
        pv_system_row_number.reshape(E * P).astype(jnp.int32),
    ).reshape(E, P, D)

    t0_arr = jnp.asarray(pv_t0_idx, jnp.int32).reshape(1, 1)

    out = pl.pallas_call(
        functools.partial(_assembly_body, T, F, D),
        grid=(E,),
        in_specs=[
            pl.BlockSpec(memory_space=pltpu.SMEM),
            pl.BlockSpec((1, 1, T), lambda e: (e, 0, 0),
                         memory_space=pltpu.SMEM),
            pl.BlockSpec((1, 1, T), lambda e: (e, 0, 0),
                         memory_space=pltpu.SMEM),
            pl.BlockSpec((1, T, F), lambda e: (e, 0, 0),
                         memory_space=pltpu.SMEM),
            pl.BlockSpec((1, 1, F), lambda e: (e, 0, 0),
                         memory_space=pltpu.SMEM),
            pl.BlockSpec((1, P, F), lambda e: (e, 0, 0)),
            pl.BlockSpec((1, P, F), lambda e: (e, 0, 0)),
            pl.BlockSpec((1, P, D), lambda e: (e, 0, 0)),
            pl.BlockSpec((1, T, P), lambda e: (e, 0, 0)),
        ],
        out_specs=pl.BlockSpec((T, C, P), lambda e: (e, 0, 0)),
        out_shape=jax.ShapeDtypeStruct((E * T, C, P), jnp.float32),
        scratch_shapes=[pltpu.VMEM((C, P), jnp.float32)],
        compiler_params=pltpu.CompilerParams(
            vmem_limit_bytes=100 * 1024 * 1024),
    )(
        t0_arr,
        pv_solar_azimuth.reshape(E, 1, T),
        pv_solar_elevation.reshape(E, 1, T),
        pv_time_utc_fourier,
        pv_time_utc_fourier_t0.reshape(E, 1, F),
        pv_y_osgb_fourier,
        pv_x_osgb_fourier,
        emb,
        pv,
    )
    return jnp.transpose(out, (0, 2, 1))


# speedup vs baseline: 1.1165x; 1.1165x over previous
import functools

import jax
import jax.numpy as jnp
from jax import lax
from jax.experimental import pallas as pl
from jax.experimental.pallas import tpu as pltpu
from jax.experimental.pallas import tpu_sc as plsc

_NUM_GSPS = 360


def _sc_embedding_gather(table, row_idx):
    (B,) = row_idx.shape
    V, D = table.shape
    info = plsc.get_sparse_core_info()
    nw = info.num_cores * info.num_subcores
    L = info.num_lanes
    b_per_w = B // nw
    mesh = plsc.VectorSubcoreMesh(core_axis_name="c", subcore_axis_name="s")

    @functools.partial(
        pl.kernel,
        mesh=mesh,
        out_type=jax.ShapeDtypeStruct((D, B), jnp.float32),
        scratch_types=[
            pltpu.VMEM((b_per_w,), jnp.int32),
            pltpu.VMEM((V, D), jnp.float32),
            pltpu.VMEM((D, b_per_w), jnp.float32),
            pltpu.SemaphoreType.DMA,
        ],
        compiler_params=pltpu.CompilerParams(
            use_tc_tiling_on_sc=False, needs_layout_passes=False),
    )
    def gather_kernel(table_hbm, idx_hbm, out_hbm, idx_v, tab_v, outT_v, sem):
        wid = lax.axis_index("s") * info.num_cores + lax.axis_index("c")
        base = wid * b_per_w
        pltpu.sync_copy(table_hbm, tab_v)
        pltpu.sync_copy(idx_hbm.at[pl.ds(base, b_per_w)], idx_v)
        for i in range(b_per_w // L):
            sl = pl.ds(i * L, L)
            idx_v[sl] = idx_v[sl] + _NUM_GSPS

        def chunk(i, carry):
            sl = pl.ds(i * L, L)
            rows = idx_v[sl]
            for d in range(D):
                outT_v[d, sl] = plsc.load_gather(
                    tab_v, [rows, jnp.full((L,), d, jnp.int32)])
            return carry

        lax.fori_loop(0, b_per_w // L, chunk, 0)
        pltpu.sync_copy(outT_v, out_hbm.at[:, pl.ds(base, b_per_w)])

    return gather_kernel(table, row_idx)


def _assembly_body(T, F, D, t0_ref, az_ref, el_ref, tf_ref, tf0_ref, y_ref,
                   x_ref, emb_ref, pv_ref, out_ref, s_ref):
    C = out_ref.shape[1]
    P = out_ref.shape[2]
    tv_lo = 1 + 2 * F
    tv_hi = 3 + 4 * F
    nan = jnp.float32(jnp.nan)

    s_ref[0:1] = jnp.zeros((1, P), jnp.float32)
    s_ref[1:1 + F] = y_ref[0]
    s_ref[1 + F:tv_lo] = x_ref[0]
    s_ref[tv_lo:tv_lo + F] = jnp.zeros((F, P), jnp.float32)
    s_ref[tv_lo + F:tv_hi - 2] = jnp.concatenate(
        [jnp.full((1, P), tf0_ref[0, 0, j], jnp.float32) for j in range(F)],
        axis=0)
    s_ref[tv_hi - 2:tv_hi] = jnp.zeros((2, P), jnp.float32)
    s_ref[tv_hi:tv_hi + D] = emb_ref[...]
    s_ref[C - 1:C] = jnp.zeros((1, P), jnp.float32)

    S = s_ref[...]
    t0v = t0_ref[0, 0]
    for t in range(T):
        out_ref[t] = S
        bad = t > t0v
        tf_rows = jnp.concatenate(
            [jnp.full((1, P), jnp.where(bad, nan, tf_ref[0, t, j]),
                      jnp.float32) for j in range(F)], axis=0)
        out_ref[t, tv_lo:tv_lo + F] = tf_rows
        azel = jnp.concatenate([
            jnp.full((1, P), jnp.where(bad, nan, az_ref[0, 0, t]),
                     jnp.float32),
            jnp.full((1, P), jnp.where(bad, nan, el_ref[0, 0, t]),
                     jnp.float32)], axis=0)
        out_ref[t, tv_hi - 2:tv_hi] = azel
        out_ref[t, C - 1:C] = jnp.where(
            bad, nan, pv_ref[0, t].reshape(1, P))


def kernel(pv, pv_solar_azimuth, pv_solar_elevation, pv_time_utc_fourier,
           pv_time_utc_fourier_t0, pv_y_osgb_fourier, pv_x_osgb_fourier,
           pv_system_row_number, pv_t0_idx, embedding_table):
    E, T, P = pv.shape
    F = pv_time_utc_fourier.shape[-1]
    D = embedding_table.shape[1]
    C = 3 + 4 * F + D + 1

    embT = _sc_embedding_gather(
        embedding_table,
        pv_system_row_number.reshape(E * P).astype(jnp.int32),
    )

    t0_arr = jnp.asarray(pv_t0_idx, jnp.int32).reshape(1, 1)

    out = pl.pallas_call(
        functools.partial(_assembly_body, T, F, D),
        grid=(E,),
        in_specs=[
            pl.BlockSpec(memory_space=pltpu.SMEM),
            pl.BlockSpec((1, 1, T), lambda e: (e, 0, 0),
                         memory_space=pltpu.SMEM),
            pl.BlockSpec((1, 1, T), lambda e: (e, 0, 0),
                         memory_space=pltpu.SMEM),
            pl.BlockSpec((1, T, F), lambda e: (e, 0, 0),
                         memory_space=pltpu.SMEM),
            pl.BlockSpec((1, 1, F), lambda e: (e, 0, 0),
                         memory_space=pltpu.SMEM),
            pl.BlockSpec((1, F, P), lambda e: (e, 0, 0)),
            pl.BlockSpec((1, F, P), lambda e: (e, 0, 0)),
            pl.BlockSpec((D, P), lambda e: (0, e)),
            pl.BlockSpec((1, T, P), lambda e: (e, 0, 0)),
        ],
        out_specs=pl.BlockSpec((T, C, P), lambda e: (e, 0, 0)),
        out_shape=jax.ShapeDtypeStruct((E * T, C, P), jnp.float32),
        scratch_shapes=[pltpu.VMEM((C, P), jnp.float32)],
        compiler_params=pltpu.CompilerParams(
            vmem_limit_bytes=100 * 1024 * 1024),
    )(
        t0_arr,
        pv_solar_azimuth.reshape(E, 1, T),
        pv_solar_elevation.reshape(E, 1, T),
        pv_time_utc_fourier,
        pv_time_utc_fourier_t0.reshape(E, 1, F),
        jnp.transpose(pv_y_osgb_fourier, (0, 2, 1)),
        jnp.transpose(pv_x_osgb_fourier, (0, 2, 1)),
        embT,
        pv,
    )
    return jnp.transpose(out, (0, 2, 1))
